# baseline (speedup 1.0000x reference)
import functools

import jax
import jax.numpy as jnp
from jax import lax
from jax.experimental import pallas as pl
from jax.experimental.pallas import tpu as pltpu
from jax.experimental.pallas import tpu_sc as plsc

_H = 16

_NC, _NS = 2, 16


def _ckv_body(x_ref, w_ref, o_ref):
    o_ref[...] = jnp.dot(x_ref[...], w_ref[...],
                         preferred_element_type=jnp.float32)


def _attn_body(xq_ref, c_ref, wq_ref, wup_ref, woutt_ref, o_ref,
               a_scr, b_scr, m_scr, *, nq, ksel, d, h, latent, scale):
    hd = d // h
    bf = jnp.bfloat16
    f32 = jnp.float32

    @pl.when(pl.program_id(0) == 0)
    def _prep():
        wq = wq_ref[...]
        wup = wup_ref[...]
        woutt = woutt_ref[...]
        for i in range(h):
            sl = slice(i * hd, (i + 1) * hd)
            a_scr[i * d:(i + 1) * d, :] = lax.dot_general(
                wq[sl], wup[sl], (((0,), (0,)), ((), ())),
                preferred_element_type=f32).astype(bf)
            b_scr[i * latent:(i + 1) * latent, :] = lax.dot_general(
                wup[d + i * hd:d + (i + 1) * hd], woutt[sl],
                (((0,), (0,)), ((), ())),
                preferred_element_type=f32).astype(bf)
        r_q = lax.broadcasted_iota(jnp.int32, (h * nq, nq * ksel), 0) % nq
        c_q = lax.broadcasted_iota(jnp.int32, (h * nq, nq * ksel), 1) // ksel
        m_scr[...] = (r_q == c_q).astype(bf)

    for j in range(xq_ref.shape[0]):
        xq = xq_ref[j].astype(bf)
        c = c_ref[j].astype(bf)
        qh = [jnp.dot(xq, a_scr[i * d:(i + 1) * d, :],
                      preferred_element_type=f32) for i in range(h)]
        qlat = jnp.concatenate(qh, axis=0).astype(bf)
        s = lax.dot_general(qlat, c, (((1,), (1,)), ((), ())),
                            preferred_element_type=f32) * scale
        e = jnp.exp(s).astype(bf) * m_scr[...]
        cp = jnp.concatenate([c, jnp.ones((nq * ksel, latent), bf)],
                             axis=1)
        o = jnp.dot(e, cp, preferred_element_type=f32)
        olat = (o[:, :latent] / o[:, latent:latent + 1]).astype(bf)
        acc = jnp.zeros((nq, d), f32)
        for i in range(h):
            acc = acc + jnp.dot(olat[i * nq:(i + 1) * nq, :],
                                b_scr[i * latent:(i + 1) * latent, :],
                                preferred_element_type=f32)
        o_ref[j] = acc


def _make_gather(total_rows, latent, nkv, rows_per_batch):
    nw = _NC * _NS
    bpw = total_rows // nw
    wpb = rows_per_batch // bpw
    mesh = plsc.VectorSubcoreMesh(core_axis_name="c", subcore_axis_name="s")

    def body(table_hbm, idx_hbm, out_hbm, idx_v, rows_v, sem):
        wid = lax.axis_index("s") * _NC + lax.axis_index("c")
        base = wid * bpw
        pltpu.sync_copy(idx_hbm.at[pl.ds(base, bpw)], idx_v)
        off = (wid // wpb) * nkv
        for i in range(bpw // 16):
            sl = pl.ds(i * 16, 16)
            idx_v[sl] = idx_v[sl] + off
        pltpu.async_copy(table_hbm.at[idx_v], rows_v, sem).wait()
        pltpu.sync_copy(rows_v, out_hbm.at[pl.ds(base, bpw)])

    return pl.kernel(
        body,
        out_type=jax.ShapeDtypeStruct((total_rows, latent), jnp.float32),
        mesh=mesh,
        scratch_types=[
            pltpu.VMEM((bpw,), jnp.int32),
            pltpu.VMEM((bpw, latent), jnp.float32),
            pltpu.SemaphoreType.DMA,
        ],
    )


def kernel(x_q, x_kv, indices, W_q, W_down, W_up, W_out):
    b, nq, d = x_q.shape
    nkv = x_kv.shape[1]
    ksel = indices.shape[2]
    latent = W_down.shape[0]
    h = _H
    scale = 1.0 / float(d // h) ** 0.5

    xkv_flat = x_kv.reshape(b * nkv, d)
    rows = 2048
    ckv_flat = pl.pallas_call(
        _ckv_body,
        grid=(b * nkv // rows,),
        in_specs=[
            pl.BlockSpec((rows, d), lambda i: (i, 0)),
            pl.BlockSpec((d, latent), lambda i: (0, 0)),
        ],
        out_specs=pl.BlockSpec((rows, latent), lambda i: (i, 0)),
        out_shape=jax.ShapeDtypeStruct((b * nkv, latent), jnp.float32),
    )(xkv_flat, W_down.T)

    idx_flat = indices.reshape(b * nq * ksel).astype(jnp.int32)
    gather = _make_gather(b * nq * ksel, latent, nkv, nq * ksel)
    c_sel_flat = gather(ckv_flat, idx_flat)
    c_sel = c_sel_flat.reshape(b, nq * ksel, latent)

    body = functools.partial(_attn_body, nq=nq, ksel=ksel, d=d, h=h,
                             latent=latent, scale=scale)
    out = pl.pallas_call(
        body,
        grid=(b // 2,),
        in_specs=[
            pl.BlockSpec((2, nq, d), lambda i: (i, 0, 0)),
            pl.BlockSpec((2, nq * ksel, latent), lambda i: (i, 0, 0)),
            pl.BlockSpec((d, d), lambda i: (0, 0)),
            pl.BlockSpec((2 * d, latent), lambda i: (0, 0)),
            pl.BlockSpec((d, d), lambda i: (0, 0)),
        ],
        out_specs=pl.BlockSpec((2, nq, d), lambda i: (i, 0, 0)),
        out_shape=jax.ShapeDtypeStruct((b, nq, d), jnp.float32),
        scratch_shapes=[
            pltpu.VMEM((h * d, latent), jnp.bfloat16),
            pltpu.VMEM((h * latent, d), jnp.bfloat16),
            pltpu.VMEM((h * nq, nq * ksel), jnp.bfloat16),
        ],
    )(x_q, c_sel, W_q, W_up, W_out.T)
    return out


# speedup vs baseline: 1.0915x; 1.0915x over previous
import functools

import jax
import jax.numpy as jnp
from jax import lax
from jax.experimental import pallas as pl
from jax.experimental.pallas import tpu as pltpu
from jax.experimental.pallas import tpu_sc as plsc

_H = 16

_NC, _NS = 2, 16


def _ckv_body(x_ref, w_ref, o_ref):
    o_ref[...] = jnp.dot(x_ref[...], w_ref[...],
                         preferred_element_type=jnp.float32)


def _attn_body(xq_ref, c_ref, wq_ref, wup_ref, woutt_ref, o_ref,
               a_scr, b_scr, m_scr, *, nq, ksel, d, h, latent, scale):
    hd = d // h
    bf = jnp.bfloat16
    f32 = jnp.float32

    @pl.when(pl.program_id(0) == 0)
    def _prep():
        wq = wq_ref[...]
        wup = wup_ref[...]
        woutt = woutt_ref[...]
        for i in range(h):
            sl = slice(i * hd, (i + 1) * hd)
            a_scr[i * d:(i + 1) * d, :] = lax.dot_general(
                wq[sl], wup[sl], (((0,), (0,)), ((), ())),
                preferred_element_type=f32).astype(bf)
            b_scr[i * latent:(i + 1) * latent, :] = lax.dot_general(
                wup[d + i * hd:d + (i + 1) * hd], woutt[sl],
                (((0,), (0,)), ((), ())),
                preferred_element_type=f32).astype(bf)
        g = nq // 2
        r_q = lax.broadcasted_iota(jnp.int32, (h * g, g * ksel), 0) % g
        c_q = lax.broadcasted_iota(jnp.int32, (h * g, g * ksel), 1) // ksel
        m_scr[...] = (r_q == c_q).astype(bf)

    ng = 2
    g = nq // ng
    for j in range(xq_ref.shape[0]):
        xq = xq_ref[j].astype(bf)
        c = c_ref[j].astype(bf)
        qh = [jnp.dot(xq, a_scr[i * d:(i + 1) * d, :],
                      preferred_element_type=f32) for i in range(h)]
        qlat = jnp.concatenate(qh, axis=0).astype(bf)
        olats = []
        for gg in range(ng):
            qlat_g = jnp.concatenate(
                [qlat[i * nq + gg * g:i * nq + gg * g + g] for i in range(h)],
                axis=0)
            c_g = c[gg * g * ksel:(gg + 1) * g * ksel]
            s = lax.dot_general(qlat_g, c_g, (((1,), (1,)), ((), ())),
                                preferred_element_type=f32) * scale
            e = jnp.exp(s).astype(bf) * m_scr[...]
            cp = jnp.concatenate([c_g, jnp.ones((g * ksel, latent), bf)],
                                 axis=1)
            o = jnp.dot(e, cp, preferred_element_type=f32)
            olats.append(
                (o[:, :latent] / o[:, latent:latent + 1]).astype(bf))
        acc = jnp.zeros((nq, d), f32)
        for i in range(h):
            olat_i = jnp.concatenate(
                [olats[gg][i * g:(i + 1) * g] for gg in range(ng)], axis=0)
            acc = acc + jnp.dot(olat_i,
                                b_scr[i * latent:(i + 1) * latent, :],
                                preferred_element_type=f32)
        o_ref[j] = acc


def _make_gather(total_rows, latent, nkv, rows_per_batch):
    nw = _NC * _NS
    bpw = total_rows // nw
    wpb = rows_per_batch // bpw
    mesh = plsc.VectorSubcoreMesh(core_axis_name="c", subcore_axis_name="s")

    def body(table_hbm, idx_hbm, out_hbm, idx_v, rows_v, sem):
        wid = lax.axis_index("s") * _NC + lax.axis_index("c")
        base = wid * bpw
        pltpu.sync_copy(idx_hbm.at[pl.ds(base, bpw)], idx_v)
        off = (wid // wpb) * nkv
        for i in range(bpw // 16):
            sl = pl.ds(i * 16, 16)
            idx_v[sl] = idx_v[sl] + off
        pltpu.async_copy(table_hbm.at[idx_v], rows_v, sem).wait()
        pltpu.sync_copy(rows_v, out_hbm.at[pl.ds(base, bpw)])

    return pl.kernel(
        body,
        out_type=jax.ShapeDtypeStruct((total_rows, latent), jnp.float32),
        mesh=mesh,
        scratch_types=[
            pltpu.VMEM((bpw,), jnp.int32),
            pltpu.VMEM((bpw, latent), jnp.float32),
            pltpu.SemaphoreType.DMA,
        ],
    )


def kernel(x_q, x_kv, indices, W_q, W_down, W_up, W_out):
    b, nq, d = x_q.shape
    nkv = x_kv.shape[1]
    ksel = indices.shape[2]
    latent = W_down.shape[0]
    h = _H
    scale = 1.0 / float(d // h) ** 0.5

    xkv_flat = x_kv.reshape(b * nkv, d)
    rows = 2048
    ckv_flat = pl.pallas_call(
        _ckv_body,
        grid=(b * nkv // rows,),
        in_specs=[
            pl.BlockSpec((rows, d), lambda i: (i, 0)),
            pl.BlockSpec((d, latent), lambda i: (0, 0)),
        ],
        out_specs=pl.BlockSpec((rows, latent), lambda i: (i, 0)),
        out_shape=jax.ShapeDtypeStruct((b * nkv, latent), jnp.float32),
    )(xkv_flat, W_down.T)

    idx_flat = indices.reshape(b * nq * ksel).astype(jnp.int32)
    gather = _make_gather(b * nq * ksel, latent, nkv, nq * ksel)
    c_sel_flat = gather(ckv_flat, idx_flat)
    c_sel = c_sel_flat.reshape(b, nq * ksel, latent)

    body = functools.partial(_attn_body, nq=nq, ksel=ksel, d=d, h=h,
                             latent=latent, scale=scale)
    out = pl.pallas_call(
        body,
        grid=(b // 2,),
        in_specs=[
            pl.BlockSpec((2, nq, d), lambda i: (i, 0, 0)),
            pl.BlockSpec((2, nq * ksel, latent), lambda i: (i, 0, 0)),
            pl.BlockSpec((d, d), lambda i: (0, 0)),
            pl.BlockSpec((2 * d, latent), lambda i: (0, 0)),
            pl.BlockSpec((d, d), lambda i: (0, 0)),
        ],
        out_specs=pl.BlockSpec((2, nq, d), lambda i: (i, 0, 0)),
        out_shape=jax.ShapeDtypeStruct((b, nq, d), jnp.float32),
        scratch_shapes=[
            pltpu.VMEM((h * d, latent), jnp.bfloat16),
            pltpu.VMEM((h * latent, d), jnp.bfloat16),
            pltpu.VMEM((h * (nq // 2), (nq // 2) * ksel), jnp.bfloat16),
        ],
    )(x_q, c_sel, W_q, W_up, W_out.T)
    return out


# speedup vs baseline: 1.0986x; 1.0064x over previous
import functools

import jax
import jax.numpy as jnp
from jax import lax
from jax.experimental import pallas as pl
from jax.experimental.pallas import tpu as pltpu
from jax.experimental.pallas import tpu_sc as plsc

_H = 16

_NC, _NS = 2, 16


def _ckv_body(x_ref, w_ref, o_ref):
    o_ref[...] = jnp.dot(x_ref[...], w_ref[...],
                         preferred_element_type=jnp.float32)


def _attn_body(xq_ref, c_ref, wq_ref, wup_ref, woutt_ref, o_ref,
               a_scr, b_scr, m_scr, *, nq, ksel, d, h, latent, scale):
    hd = d // h
    bf = jnp.bfloat16
    f32 = jnp.float32

    @pl.when(pl.program_id(0) == 0)
    def _prep():
        wq = wq_ref[...]
        wup = wup_ref[...]
        woutt = woutt_ref[...]
        for i in range(h):
            sl = slice(i * hd, (i + 1) * hd)
            a_scr[i * d:(i + 1) * d, :] = lax.dot_general(
                wq[sl], wup[sl], (((0,), (0,)), ((), ())),
                preferred_element_type=f32).astype(bf)
            b_scr[i * latent:(i + 1) * latent, :] = lax.dot_general(
                wup[d + i * hd:d + (i + 1) * hd], woutt[sl],
                (((0,), (0,)), ((), ())),
                preferred_element_type=f32).astype(bf)
        g = nq // 4
        r_q = lax.broadcasted_iota(jnp.int32, (h * g, g * ksel), 0) % g
        c_q = lax.broadcasted_iota(jnp.int32, (h * g, g * ksel), 1) // ksel
        m_scr[...] = (r_q == c_q).astype(bf)

    ng = 4
    g = nq // ng
    for j in range(xq_ref.shape[0]):
        xq = xq_ref[j].astype(bf)
        c = c_ref[j].astype(bf)
        qh = [jnp.dot(xq, a_scr[i * d:(i + 1) * d, :],
                      preferred_element_type=f32) for i in range(h)]
        qlat = jnp.concatenate(qh, axis=0).astype(bf)
        olats = []
        for gg in range(ng):
            qlat_g = jnp.concatenate(
                [qlat[i * nq + gg * g:i * nq + gg * g + g] for i in range(h)],
                axis=0)
            c_g = c[gg * g * ksel:(gg + 1) * g * ksel]
            s = lax.dot_general(qlat_g, c_g, (((1,), (1,)), ((), ())),
                                preferred_element_type=f32) * scale
            e = jnp.exp(s).astype(bf) * m_scr[...]
            cp = jnp.concatenate([c_g, jnp.ones((g * ksel, latent), bf)],
                                 axis=1)
            o = jnp.dot(e, cp, preferred_element_type=f32)
            olats.append(
                (o[:, :latent] / o[:, latent:latent + 1]).astype(bf))
        acc = jnp.zeros((nq, d), f32)
        for i in range(h):
            olat_i = jnp.concatenate(
                [olats[gg][i * g:(i + 1) * g] for gg in range(ng)], axis=0)
            acc = acc + jnp.dot(olat_i,
                                b_scr[i * latent:(i + 1) * latent, :],
                                preferred_element_type=f32)
        o_ref[j] = acc


def _make_gather(total_rows, latent, nkv, rows_per_batch):
    nw = _NC * _NS
    bpw = total_rows // nw
    wpb = rows_per_batch // bpw
    mesh = plsc.VectorSubcoreMesh(core_axis_name="c", subcore_axis_name="s")

    def body(table_hbm, idx_hbm, out_hbm, idx_v, rows_v, sem):
        wid = lax.axis_index("s") * _NC + lax.axis_index("c")
        base = wid * bpw
        pltpu.sync_copy(idx_hbm.at[pl.ds(base, bpw)], idx_v)
        off = (wid // wpb) * nkv
        for i in range(bpw // 16):
            sl = pl.ds(i * 16, 16)
            idx_v[sl] = idx_v[sl] + off
        pltpu.async_copy(table_hbm.at[idx_v], rows_v, sem).wait()
        pltpu.sync_copy(rows_v, out_hbm.at[pl.ds(base, bpw)])

    return pl.kernel(
        body,
        out_type=jax.ShapeDtypeStruct((total_rows, latent), jnp.float32),
        mesh=mesh,
        scratch_types=[
            pltpu.VMEM((bpw,), jnp.int32),
            pltpu.VMEM((bpw, latent), jnp.float32),
            pltpu.SemaphoreType.DMA,
        ],
    )


def kernel(x_q, x_kv, indices, W_q, W_down, W_up, W_out):
    b, nq, d = x_q.shape
    nkv = x_kv.shape[1]
    ksel = indices.shape[2]
    latent = W_down.shape[0]
    h = _H
    scale = 1.0 / float(d // h) ** 0.5

    xkv_flat = x_kv.reshape(b * nkv, d)
    rows = 2048
    ckv_flat = pl.pallas_call(
        _ckv_body,
        grid=(b * nkv // rows,),
        in_specs=[
            pl.BlockSpec((rows, d), lambda i: (i, 0)),
            pl.BlockSpec((d, latent), lambda i: (0, 0)),
        ],
        out_specs=pl.BlockSpec((rows, latent), lambda i: (i, 0)),
        out_shape=jax.ShapeDtypeStruct((b * nkv, latent), jnp.float32),
    )(xkv_flat, W_down.T)

    idx_flat = indices.reshape(b * nq * ksel).astype(jnp.int32)
    gather = _make_gather(b * nq * ksel, latent, nkv, nq * ksel)
    c_sel_flat = gather(ckv_flat, idx_flat)
    c_sel = c_sel_flat.reshape(b, nq * ksel, latent)

    body = functools.partial(_attn_body, nq=nq, ksel=ksel, d=d, h=h,
                             latent=latent, scale=scale)
    out = pl.pallas_call(
        body,
        grid=(b // 2,),
        in_specs=[
            pl.BlockSpec((2, nq, d), lambda i: (i, 0, 0)),
            pl.BlockSpec((2, nq * ksel, latent), lambda i: (i, 0, 0)),
            pl.BlockSpec((d, d), lambda i: (0, 0)),
            pl.BlockSpec((2 * d, latent), lambda i: (0, 0)),
            pl.BlockSpec((d, d), lambda i: (0, 0)),
        ],
        out_specs=pl.BlockSpec((2, nq, d), lambda i: (i, 0, 0)),
        out_shape=jax.ShapeDtypeStruct((b, nq, d), jnp.float32),
        scratch_shapes=[
            pltpu.VMEM((h * d, latent), jnp.bfloat16),
            pltpu.VMEM((h * latent, d), jnp.bfloat16),
            pltpu.VMEM((h * (nq // 4), (nq // 4) * ksel), jnp.bfloat16),
        ],
    )(x_q, c_sel, W_q, W_up, W_out.T)
    return out


# speedup vs baseline: 1.1735x; 1.0682x over previous
import functools

import jax
import jax.numpy as jnp
from jax import lax
from jax.experimental import pallas as pl
from jax.experimental.pallas import tpu as pltpu
from jax.experimental.pallas import tpu_sc as plsc

_H = 16

_NC, _NS = 2, 16


def _ckv_body(x_ref, w_ref, o_ref):
    o_ref[...] = jnp.dot(x_ref[...], w_ref[...],
                         preferred_element_type=jnp.float32)


def _attn_body(xq_ref, c_ref, wq_ref, wup_ref, woutt_ref, o_ref,
               a_scr, b_scr, m_scr, *, nq, ksel, d, h, latent, scale):
    hd = d // h
    bf = jnp.bfloat16
    f32 = jnp.float32

    @pl.when(pl.program_id(0) == 0)
    def _prep():
        wq = wq_ref[...]
        wup = wup_ref[...]
        woutt = woutt_ref[...]
        for i in range(h):
            sl = slice(i * hd, (i + 1) * hd)
            a_scr[i * d:(i + 1) * d, :] = lax.dot_general(
                wq[sl], wup[sl], (((0,), (0,)), ((), ())),
                preferred_element_type=f32).astype(bf)
            b_scr[i * latent:(i + 1) * latent, :] = lax.dot_general(
                wup[d + i * hd:d + (i + 1) * hd], woutt[sl],
                (((0,), (0,)), ((), ())),
                preferred_element_type=f32).astype(bf)
        g = nq // 4
        r_q = lax.broadcasted_iota(jnp.int32, (h * g, g * ksel), 0) % g
        c_q = lax.broadcasted_iota(jnp.int32, (h * g, g * ksel), 1) // ksel
        m_scr[...] = (r_q == c_q).astype(bf)

    ng = 4
    g = nq // ng
    nb = xq_ref.shape[0]
    xq_all = jnp.concatenate([xq_ref[j] for j in range(nb)],
                             axis=0).astype(bf)
    qh = [jnp.dot(xq_all, a_scr[i * d:(i + 1) * d, :],
                  preferred_element_type=f32) for i in range(h)]
    qlat = jnp.concatenate(qh, axis=0).astype(bf)
    olats = {}
    for j in range(nb):
        c = c_ref[j].astype(bf)
        for gg in range(ng):
            base = j * nq + gg * g
            qlat_g = jnp.concatenate(
                [qlat[i * nb * nq + base:i * nb * nq + base + g]
                 for i in range(h)], axis=0)
            c_g = c[gg * g * ksel:(gg + 1) * g * ksel]
            s = lax.dot_general(qlat_g, c_g, (((1,), (1,)), ((), ())),
                                preferred_element_type=f32) * scale
            e = jnp.exp(s).astype(bf) * m_scr[...]
            cp = jnp.concatenate([c_g, jnp.ones((g * ksel, latent), bf)],
                                 axis=1)
            o = jnp.dot(e, cp, preferred_element_type=f32)
            olats[j, gg] = (o[:, :latent] /
                            o[:, latent:latent + 1]).astype(bf)
    acc = jnp.zeros((nb * nq, d), f32)
    for i in range(h):
        olat_i = jnp.concatenate(
            [olats[j, gg][i * g:(i + 1) * g]
             for j in range(nb) for gg in range(ng)], axis=0)
        acc = acc + jnp.dot(olat_i, b_scr[i * latent:(i + 1) * latent, :],
                            preferred_element_type=f32)
    for j in range(nb):
        o_ref[j] = acc[j * nq:(j + 1) * nq]


def _make_gather(total_rows, latent, nkv, rows_per_batch):
    nw = _NC * _NS
    bpw = total_rows // nw
    wpb = rows_per_batch // bpw
    mesh = plsc.VectorSubcoreMesh(core_axis_name="c", subcore_axis_name="s")

    def body(table_hbm, idx_hbm, out_hbm, idx_v, rows_v, sem):
        wid = lax.axis_index("s") * _NC + lax.axis_index("c")
        base = wid * bpw
        pltpu.sync_copy(idx_hbm.at[pl.ds(base, bpw)], idx_v)
        off = (wid // wpb) * nkv
        for i in range(bpw // 16):
            sl = pl.ds(i * 16, 16)
            idx_v[sl] = idx_v[sl] + off
        pltpu.async_copy(table_hbm.at[idx_v], rows_v, sem).wait()
        pltpu.sync_copy(rows_v, out_hbm.at[pl.ds(base, bpw)])

    return pl.kernel(
        body,
        out_type=jax.ShapeDtypeStruct((total_rows, latent), jnp.float32),
        mesh=mesh,
        scratch_types=[
            pltpu.VMEM((bpw,), jnp.int32),
            pltpu.VMEM((bpw, latent), jnp.float32),
            pltpu.SemaphoreType.DMA,
        ],
    )


def kernel(x_q, x_kv, indices, W_q, W_down, W_up, W_out):
    b, nq, d = x_q.shape
    nkv = x_kv.shape[1]
    ksel = indices.shape[2]
    latent = W_down.shape[0]
    h = _H
    scale = 1.0 / float(d // h) ** 0.5

    xkv_flat = x_kv.reshape(b * nkv, d)
    rows = 2048
    ckv_flat = pl.pallas_call(
        _ckv_body,
        grid=(b * nkv // rows,),
        in_specs=[
            pl.BlockSpec((rows, d), lambda i: (i, 0)),
            pl.BlockSpec((d, latent), lambda i: (0, 0)),
        ],
        out_specs=pl.BlockSpec((rows, latent), lambda i: (i, 0)),
        out_shape=jax.ShapeDtypeStruct((b * nkv, latent), jnp.float32),
    )(xkv_flat, W_down.T)

    idx_flat = indices.reshape(b * nq * ksel).astype(jnp.int32)
    gather = _make_gather(b * nq * ksel, latent, nkv, nq * ksel)
    c_sel_flat = gather(ckv_flat, idx_flat)
    c_sel = c_sel_flat.reshape(b, nq * ksel, latent)

    body = functools.partial(_attn_body, nq=nq, ksel=ksel, d=d, h=h,
                             latent=latent, scale=scale)
    out = pl.pallas_call(
        body,
        grid=(b // 2,),
        in_specs=[
            pl.BlockSpec((2, nq, d), lambda i: (i, 0, 0)),
            pl.BlockSpec((2, nq * ksel, latent), lambda i: (i, 0, 0)),
            pl.BlockSpec((d, d), lambda i: (0, 0)),
            pl.BlockSpec((2 * d, latent), lambda i: (0, 0)),
            pl.BlockSpec((d, d), lambda i: (0, 0)),
        ],
        out_specs=pl.BlockSpec((2, nq, d), lambda i: (i, 0, 0)),
        out_shape=jax.ShapeDtypeStruct((b, nq, d), jnp.float32),
        scratch_shapes=[
            pltpu.VMEM((h * d, latent), jnp.bfloat16),
            pltpu.VMEM((h * latent, d), jnp.bfloat16),
            pltpu.VMEM((h * (nq // 4), (nq // 4) * ksel), jnp.bfloat16),
        ],
    )(x_q, c_sel, W_q, W_up, W_out.T)
    return out


# speedup vs baseline: 1.2398x; 1.0565x over previous
import functools

import jax
import jax.numpy as jnp
from jax import lax
from jax.experimental import pallas as pl
from jax.experimental.pallas import tpu as pltpu
from jax.experimental.pallas import tpu_sc as plsc

_H = 16

_NC, _NS = 2, 16


def _ckv_body(x_ref, w_ref, o_ref):
    o_ref[...] = jnp.dot(x_ref[...], w_ref[...],
                         preferred_element_type=jnp.float32)


def _attn_body(xq_ref, c_ref, wq_ref, wup_ref, woutt_ref, o_ref,
               a_scr, b_scr, m_scr, *, nq, ksel, d, h, latent, scale):
    hd = d // h
    bf = jnp.bfloat16
    f32 = jnp.float32

    @pl.when(pl.program_id(0) == 0)
    def _prep():
        wq = wq_ref[...]
        wup = wup_ref[...]
        woutt = woutt_ref[...]
        for i in range(h):
            sl = slice(i * hd, (i + 1) * hd)
            a_scr[i * d:(i + 1) * d, :] = lax.dot_general(
                wq[sl], wup[sl], (((0,), (0,)), ((), ())),
                preferred_element_type=f32).astype(bf)
            b_scr[i * latent:(i + 1) * latent, :] = lax.dot_general(
                wup[d + i * hd:d + (i + 1) * hd], woutt[sl],
                (((0,), (0,)), ((), ())),
                preferred_element_type=f32).astype(bf)
        g = nq // 4
        r_q = lax.broadcasted_iota(jnp.int32, (h * g, g * ksel), 0) % g
        c_q = lax.broadcasted_iota(jnp.int32, (h * g, g * ksel), 1) // ksel
        m_scr[...] = (r_q == c_q).astype(bf)

    ng = 4
    g = nq // ng
    nb = xq_ref.shape[0]
    xq_all = jnp.concatenate([xq_ref[j] for j in range(nb)],
                             axis=0).astype(bf)
    qh = [jnp.dot(xq_all, a_scr[i * d:(i + 1) * d, :],
                  preferred_element_type=f32) for i in range(h)]
    qlat = jnp.concatenate(qh, axis=0).astype(bf)
    olats = {}
    for j in range(nb):
        c = c_ref[j].astype(bf)
        for gg in range(ng):
            base = j * nq + gg * g
            qlat_g = jnp.concatenate(
                [qlat[i * nb * nq + base:i * nb * nq + base + g]
                 for i in range(h)], axis=0)
            c_g = c[gg * g * ksel:(gg + 1) * g * ksel]
            s = lax.dot_general(qlat_g, c_g, (((1,), (1,)), ((), ())),
                                preferred_element_type=f32) * scale
            e = jnp.exp(s).astype(bf) * m_scr[...]
            cp = jnp.concatenate([c_g, jnp.ones((g * ksel, latent), bf)],
                                 axis=1)
            o = jnp.dot(e, cp, preferred_element_type=f32)
            olats[j, gg] = (o[:, :latent] /
                            o[:, latent:latent + 1]).astype(bf)
    acc = jnp.zeros((nb * nq, d), f32)
    for i in range(h):
        olat_i = jnp.concatenate(
            [olats[j, gg][i * g:(i + 1) * g]
             for j in range(nb) for gg in range(ng)], axis=0)
        acc = acc + jnp.dot(olat_i, b_scr[i * latent:(i + 1) * latent, :],
                            preferred_element_type=f32)
    for j in range(nb):
        o_ref[j] = acc[j * nq:(j + 1) * nq]


def _make_gather(total_rows, latent, nkv, rows_per_batch):
    nw = _NC * _NS
    bpw = total_rows // nw
    wpb = rows_per_batch // bpw
    mesh = plsc.VectorSubcoreMesh(core_axis_name="c", subcore_axis_name="s")

    def body(table_hbm, idx_hbm, out_hbm, idx_v, rows_v, sem):
        wid = lax.axis_index("s") * _NC + lax.axis_index("c")
        base = wid * bpw
        pltpu.sync_copy(idx_hbm.at[pl.ds(base, bpw)], idx_v)
        off = (wid // wpb) * nkv
        for i in range(bpw // 16):
            sl = pl.ds(i * 16, 16)
            idx_v[sl] = idx_v[sl] + off
        pltpu.async_copy(table_hbm.at[idx_v], rows_v, sem).wait()
        pltpu.sync_copy(rows_v, out_hbm.at[pl.ds(base, bpw)])

    return pl.kernel(
        body,
        out_type=jax.ShapeDtypeStruct((total_rows, latent), jnp.float32),
        mesh=mesh,
        scratch_types=[
            pltpu.VMEM((bpw,), jnp.int32),
            pltpu.VMEM((bpw, latent), jnp.float32),
            pltpu.SemaphoreType.DMA,
        ],
    )


def kernel(x_q, x_kv, indices, W_q, W_down, W_up, W_out):
    b, nq, d = x_q.shape
    nkv = x_kv.shape[1]
    ksel = indices.shape[2]
    latent = W_down.shape[0]
    h = _H
    scale = 1.0 / float(d // h) ** 0.5

    xkv_flat = x_kv.reshape(b * nkv, d)
    rows = 2048
    ckv_flat = pl.pallas_call(
        _ckv_body,
        grid=(b * nkv // rows,),
        in_specs=[
            pl.BlockSpec((rows, d), lambda i: (i, 0)),
            pl.BlockSpec((d, latent), lambda i: (0, 0)),
        ],
        out_specs=pl.BlockSpec((rows, latent), lambda i: (i, 0)),
        out_shape=jax.ShapeDtypeStruct((b * nkv, latent), jnp.float32),
    )(xkv_flat, W_down.T)

    idx_flat = indices.reshape(b * nq * ksel).astype(jnp.int32)
    gather = _make_gather(b * nq * ksel, latent, nkv, nq * ksel)
    c_sel_flat = gather(ckv_flat, idx_flat)
    c_sel = c_sel_flat.reshape(b, nq * ksel, latent)

    body = functools.partial(_attn_body, nq=nq, ksel=ksel, d=d, h=h,
                             latent=latent, scale=scale)
    out = pl.pallas_call(
        body,
        grid=(b // 4,),
        in_specs=[
            pl.BlockSpec((4, nq, d), lambda i: (i, 0, 0)),
            pl.BlockSpec((4, nq * ksel, latent), lambda i: (i, 0, 0)),
            pl.BlockSpec((d, d), lambda i: (0, 0)),
            pl.BlockSpec((2 * d, latent), lambda i: (0, 0)),
            pl.BlockSpec((d, d), lambda i: (0, 0)),
        ],
        out_specs=pl.BlockSpec((4, nq, d), lambda i: (i, 0, 0)),
        out_shape=jax.ShapeDtypeStruct((b, nq, d), jnp.float32),
        scratch_shapes=[
            pltpu.VMEM((h * d, latent), jnp.bfloat16),
            pltpu.VMEM((h * latent, d), jnp.bfloat16),
            pltpu.VMEM((h * (nq // 4), (nq // 4) * ksel), jnp.bfloat16),
        ],
    )(x_q, c_sel, W_q, W_up, W_out.T)
    return out
